# baseline (speedup 1.0000x reference)
import functools

import jax
import jax.numpy as jnp
from jax import lax
from jax.experimental import pallas as pl
from jax.experimental.pallas import tpu as pltpu
from jax.experimental.pallas import tpu_sc as plsc

N = 10000
B = 8
IN_F = 9
OUT_F = 128
HID = 128

NN = N + B
NP = 10016
ET = 340000
ETP = 344064
NW = 32
EPW = ETP // NW
CH = 256
NCH = EPW // CH
BK = 2048
NBK = ETP // BK
RPB = BK // 128


def _swish(x):
    return x * jax.nn.sigmoid(x)


def _k0_body(oh_ref, am_ref, m_ref, m2p_ref, pos_ref, wemb_ref, bemb_ref,
             he_ref, p_ref, cnt2_ref):
    oh = oh_ref[...]
    am = am_ref[...]
    m = m_ref[...]
    m2p = m2p_ref[...]
    pos = pos_ref[...]
    h0 = jnp.dot(oh, am)
    cnt = jnp.maximum(jnp.sum(m, axis=1, keepdims=True), 1.0)
    h_suf = jnp.dot(m, h0) / cnt
    p_suf = jnp.dot(m, pos) / cnt
    hcat = h0 + jnp.dot(m2p, h_suf)
    pcat = pos + jnp.dot(m2p, p_suf)
    he_ref[...] = jnp.dot(hcat, wemb_ref[...]) + bemb_ref[...]
    p_ref[...] = pcat
    cnt2_ref[...] = cnt + 1.0


def _k1_body(h_ref, p_ref, w1a_ref, w1b_ref, b1_ref, t1_ref, t2_ref):
    h = h_ref[...]
    t1_ref[...] = jnp.dot(h, w1a_ref[...]) + b1_ref[...]
    t2_ref[...] = jnp.dot(h, w1b_ref[...])
    del p_ref


def _expand_col(colM):
    r = lax.broadcasted_iota(jnp.int32, (BK, RPB), 0) // 128
    k = lax.broadcasted_iota(jnp.int32, (BK, RPB), 1)
    o1 = (r == k).astype(jnp.float32)
    g = jnp.dot(o1, colM)
    lm = lax.broadcasted_iota(jnp.int32, (BK, 128), 0) % 128
    ll = lax.broadcasted_iota(jnp.int32, (BK, 128), 1)
    sel = (ll == lm).astype(jnp.float32)
    return jnp.sum(g * sel, axis=1, keepdims=True)


def _compress_col(col):
    r = lax.broadcasted_iota(jnp.int32, (BK, RPB), 0) // 128
    k = lax.broadcasted_iota(jnp.int32, (BK, RPB), 1)
    o1 = (r == k).astype(jnp.float32)
    lm = lax.broadcasted_iota(jnp.int32, (BK, 128), 0) % 128
    ll = lax.broadcasted_iota(jnp.int32, (BK, 128), 1)
    sel = (ll == lm).astype(jnp.float32)
    return jnp.dot(o1.T, sel * col)


def _k3_body(xi_ref, xj_ref, distM_ref, w1c_ref, m0_ref, s1_ref, s2_ref):
    i = pl.program_id(0)
    dist = _expand_col(distM_ref[...])
    z = xi_ref[...] + xj_ref[...] + dist * w1c_ref[...]
    m0 = z * jax.nn.sigmoid(z)
    m0_ref[...] = m0
    gidx = i * BK + lax.broadcasted_iota(jnp.int32, (BK, 1), 0)
    mask = (gidx < ET).astype(jnp.float32)
    mm = m0 * mask

    @pl.when(i == 0)
    def _():
        s1_ref[...] = jnp.zeros_like(s1_ref)
        s2_ref[...] = jnp.zeros_like(s2_ref)

    s1_ref[...] += jnp.sum(mm, axis=0, keepdims=True)
    s2_ref[...] += jnp.sum(mm * m0, axis=0, keepdims=True)


def _k4_body(m0_ref, dvxM_ref, dvyM_ref, dvzM_ref, s1_ref, s2_ref,
             g_ref, bb_ref, w2_ref, b2_ref, wi_ref, bi_ref,
             wp1_ref, bp1_ref, wp2_ref, bp2_ref,
             msg_ref, pmxM_ref, pmyM_ref, pmzM_ref):
    mu = s1_ref[...] / ET
    var = s2_ref[...] / ET - mu * mu
    mn = (m0_ref[...] - mu) / jnp.sqrt(var + 1e-5) * g_ref[...] + bb_ref[...]
    m1 = _swish(jnp.dot(mn, w2_ref[...]) + b2_ref[...])
    t = jnp.sum(m1 * wi_ref[...], axis=1, keepdims=True) + bi_ref[...]
    m2 = jax.nn.sigmoid(t) * m1
    u = _swish(jnp.dot(m2, wp1_ref[...]) + bp1_ref[...])
    q = jnp.sum(u * wp2_ref[...], axis=1, keepdims=True) + bp2_ref[...]
    msg_ref[...] = m2
    pmxM_ref[...] = _compress_col(_expand_col(dvxM_ref[...]) * q)
    pmyM_ref[...] = _compress_col(_expand_col(dvyM_ref[...]) * q)
    pmzM_ref[...] = _compress_col(_expand_col(dvzM_ref[...]) * q)


NB = 2504
NNB = NP // NB


def _k6a_body(h_ref, p_ref, a0_ref, a1_ref, s0_ref, s16_1_ref, tagsf_ref,
              m2t_ref, wu1a_ref, wu1b_ref, bu1_ref, wu2_ref, bu2_ref,
              h2_ref, pn_ref, sgm_ref, sgv_ref):
    i = pl.program_id(0)
    aggm = a0_ref[...] + a1_ref[...]
    a16 = s0_ref[...] + s16_1_ref[...]
    deg = jnp.maximum(a16[:, 3:4], 1.0)
    aggm = aggm / deg
    pupd = a16[:, 0:4] / deg
    colmask = (lax.broadcasted_iota(jnp.int32, (NB, 4), 1) < 3).astype(jnp.float32)
    h = h_ref[...]
    pn_ref[...] = p_ref[...] + pupd * colmask * tagsf_ref[...]
    u = _swish(jnp.dot(h, wu1a_ref[...]) + jnp.dot(aggm, wu1b_ref[...]) + bu1_ref[...])
    h2 = h + _swish(jnp.dot(u, wu2_ref[...]) + bu2_ref[...])
    h2_ref[...] = h2

    @pl.when(i == 0)
    def _():
        sgm_ref[...] = jnp.zeros_like(sgm_ref)
        sgv_ref[...] = jnp.zeros_like(sgv_ref)

    m2t = m2t_ref[...]
    dn = (((0,), (0,)), ((), ()))
    sgm_ref[...] += lax.dot_general(m2t, h2, dn)
    sgv_ref[...] += lax.dot_general(m2t, h2 * h2, dn)


def _k6b_body(h2_ref, m2t_ref, sgm_ref, sgv_ref, cnt2_ref, hn_ref):
    i = pl.program_id(0)
    cnt2 = cnt2_ref[...]
    gm = sgm_ref[...] / cnt2
    gv = sgv_ref[...] / cnt2 - gm * gm
    m2t = m2t_ref[...]
    bgm = jnp.dot(m2t, gm)
    bgv = jnp.dot(m2t, gv)
    h2 = h2_ref[...]
    gidx = i * NB + lax.broadcasted_iota(jnp.int32, (NB, 1), 0)
    valid = (gidx < NN).astype(jnp.float32)
    hn_ref[...] = (h2 - bgm) / jnp.sqrt(jnp.maximum(bgv, 0.0) + 1e-5) * valid


def _k7_body(e_ref, w1_ref, b1_ref, w2_ref, b2_ref, w3_ref, b3_ref, o_ref):
    e = e_ref[...]
    e = _swish(jnp.dot(e, w1_ref[...]) + b1_ref[...])
    e = _swish(jnp.dot(e, w2_ref[...]) + b2_ref[...])
    o_ref[...] = jnp.dot(e, w3_ref[...]) + b3_ref[...]


_f32 = jnp.float32


def _sds(shape):
    return jax.ShapeDtypeStruct(shape, _f32)


def kernel(atomic_numbers, pos, edge_index, cell_offsets, tags, batch, atom_map, params):
    idt = edge_index.dtype
    edge_i = N + batch
    edge_j = jnp.arange(N, dtype=idt)
    src = jnp.concatenate([edge_index[0], edge_i, edge_j])
    dst = jnp.concatenate([edge_index[1], edge_j, edge_i])
    pad_e = ETP - ET
    dst_p = jnp.concatenate([dst, jnp.full((pad_e,), NN, idt)]).astype(jnp.int32)
    src_p = jnp.concatenate([src, jnp.full((pad_e,), NN, idt)]).astype(jnp.int32)
    co = jnp.concatenate([cell_offsets, jnp.zeros((2 * N, 3), _f32)], 0)
    co_p = jnp.concatenate([co, jnp.zeros((pad_e, 3), _f32)], 0)
    cox, coy, coz = co_p[:, 0], co_p[:, 1], co_p[:, 2]

    oh = (atomic_numbers[:, None] == jnp.arange(104)[None, :]).astype(_f32)
    oh = jnp.concatenate([oh, jnp.zeros((NP - N, 104), _f32)], 0)
    amp = jnp.zeros((104, 16), _f32).at[:101, :IN_F].set(atom_map)
    gids = jnp.arange(B, dtype=batch.dtype)
    m_real = (batch[None, :] == gids[:, None]).astype(_f32)
    m_mat = jnp.concatenate([m_real, jnp.zeros((B, NP - N), _f32)], 1)
    phant = jnp.concatenate(
        [jnp.zeros((B, N), _f32), jnp.eye(B, dtype=_f32),
         jnp.zeros((B, NP - NN), _f32)], 1)
    m2_mat = m_mat + phant
    m2p = phant.T
    pos_p = jnp.zeros((NP, 4), _f32).at[:N, :3].set(pos)
    tagsf = jnp.concatenate(
        [(tags == 2).astype(_f32), jnp.ones((B,), _f32),
         jnp.zeros((NP - NN,), _f32)])[:, None]
    wemb, bemb = params['emb']
    wemb_p = jnp.zeros((16, 128), _f32).at[:IN_F].set(wemb)

    he, p, cnt2 = pl.pallas_call(
        _k0_body,
        out_shape=[_sds((NP, 128)), _sds((NP, 4)), _sds((B, 1))],
    )(oh, amp, m_mat, m2p, pos_p, wemb_p, bemb[None, :])

    h = he
    for lp in params['layers']:
        w1, b1 = lp['msg1']
        w1a, w1b, w1c = w1[:128], w1[128:256], w1[256:257]
        t1, t2 = pl.pallas_call(
            _k1_body,
            out_shape=[_sds((NP, 128)), _sds((NP, 128))],
        )(h, p, w1a, w1b, b1[None, :])

        xi = t1[dst_p]
        xj = t2[src_p]
        dvx = pos_p[dst_p, 0] - pos_p[src_p, 0] + cox
        dvy = pos_p[dst_p, 1] - pos_p[src_p, 1] + coy
        dvz = pos_p[dst_p, 2] - pos_p[src_p, 2] + coz
        dist = dvx * dvx + dvy * dvy + dvz * dvz
        distM = dist.reshape(2688, 128)
        dvxM = dvx.reshape(2688, 128)
        dvyM = dvy.reshape(2688, 128)
        dvzM = dvz.reshape(2688, 128)

        espec = pl.BlockSpec((BK, 128), lambda i: (i, 0))
        cspec = pl.BlockSpec((RPB, 128), lambda i: (i, 0))
        wspec = pl.BlockSpec((1, 128), lambda i: (0, 0))
        m0, s1, s2 = pl.pallas_call(
            _k3_body,
            grid=(NBK,),
            in_specs=[espec, espec, cspec, wspec],
            out_specs=[espec, wspec, wspec],
            out_shape=[_sds((ETP, 128)), _sds((1, 128)), _sds((1, 128))],
        )(xi, xj, distM, w1c)

        g, bb = lp['bn']
        w2, b2 = lp['msg2']
        wi, bi = lp['inf']
        wp1, bp1 = lp['pos1']
        wp2, bp2 = lp['pos2']
        w128 = pl.BlockSpec((128, 128), lambda i: (0, 0))
        s11 = pl.BlockSpec((1, 1), lambda i: (0, 0))
        msgM, pmxM, pmyM, pmzM = pl.pallas_call(
            _k4_body,
            grid=(NBK,),
            in_specs=[espec, cspec, cspec, cspec, wspec, wspec,
                      wspec, wspec, w128, wspec, wspec, s11,
                      w128, wspec, wspec, s11],
            out_specs=[espec, cspec, cspec, cspec],
            out_shape=[_sds((ETP, 128)), _sds((2688, 128)),
                       _sds((2688, 128)), _sds((2688, 128))],
        )(m0, dvxM, dvyM, dvzM, s1, s2, g[None, :], bb[None, :],
          w2, b2[None, :], wi.T, bi[None, :], wp1, bp1[None, :],
          wp2.T, bp2[None, :])

        aggM0 = jax.ops.segment_sum(msgM, dst_p, NP)
        aggM1 = jnp.zeros_like(aggM0)
        ones_col = jnp.ones((ETP,), _f32)
        rows16 = jnp.stack([pmxM.reshape(-1), pmyM.reshape(-1),
                            pmzM.reshape(-1), ones_col], 1)
        rows16 = jnp.concatenate([rows16, jnp.zeros((ETP, 12), _f32)], 1)
        agg16_0 = jax.ops.segment_sum(rows16, dst_p, NP)
        agg16_1 = jnp.zeros_like(agg16_0)

        wu1, bu1 = lp['upd1']
        wu2, bu2 = lp['upd2']
        nspec = pl.BlockSpec((NB, 128), lambda i: (i, 0))
        n4spec = pl.BlockSpec((NB, 4), lambda i: (i, 0))
        n16spec = pl.BlockSpec((NB, 16), lambda i: (i, 0))
        n1spec = pl.BlockSpec((NB, 1), lambda i: (i, 0))
        g8spec = pl.BlockSpec((8, NB), lambda i: (0, i))
        g8tspec = pl.BlockSpec((NB, 8), lambda i: (i, 0))
        s8spec = pl.BlockSpec((8, 128), lambda i: (0, 0))
        s81spec = pl.BlockSpec((8, 1), lambda i: (0, 0))
        w128s = pl.BlockSpec((128, 128), lambda i: (0, 0))
        w1s = pl.BlockSpec((1, 128), lambda i: (0, 0))
        h2, p, sgm, sgv = pl.pallas_call(
            _k6a_body,
            grid=(NNB,),
            in_specs=[nspec, n4spec, nspec, nspec, n16spec, n16spec,
                      n1spec, g8tspec, w128s, w128s, w1s, w128s, w1s],
            out_specs=[nspec, n4spec, s8spec, s8spec],
            out_shape=[_sds((NP, 128)), _sds((NP, 4)),
                       _sds((8, 128)), _sds((8, 128))],
        )(h, p, aggM0, aggM1, agg16_0, agg16_1, tagsf, m2_mat.T,
          wu1[:128], wu1[128:], bu1[None, :], wu2, bu2[None, :])
        h = pl.pallas_call(
            _k6b_body,
            grid=(NNB,),
            in_specs=[nspec, g8tspec, s8spec, s8spec, s81spec],
            out_specs=nspec,
            out_shape=_sds((NP, 128)),
        )(h2, m2_mat.T, sgm, sgv, cnt2)

    e = h[N:NN]
    (w1, b1), (w2, b2), (w3, b3) = params['e1'], params['e2'], params['e3']
    return pl.pallas_call(
        _k7_body,
        out_shape=jax.ShapeDtypeStruct((B, 1), _f32),
    )(e, w1, b1[None, :], w2, b2[None, :], w3, b3[None, :])


# speedup vs baseline: 2.8609x; 2.8609x over previous
import functools

import jax
import jax.numpy as jnp
from jax import lax
from jax.experimental import pallas as pl
from jax.experimental.pallas import tpu as pltpu
from jax.experimental.pallas import tpu_sc as plsc

N = 10000
B = 8
IN_F = 9
OUT_F = 128
HID = 128

NN = N + B
NP = 10016
ET = 340000
ETP = 344064
NW = 32
HALF = NP // 2
HALF_P = 5120
EPW = ETP // NW
CH = 256
NCH = EPW // CH
BK = 2048
NBK = ETP // BK
RPB = BK // 128


def _swish(x):
    return x * jax.nn.sigmoid(x)


_f32 = jnp.float32


def _sds(shape):
    return jax.ShapeDtypeStruct(shape, _f32)


def _k0_body(oh_ref, am_ref, m_ref, m2p_ref, pos_ref, wemb_ref, bemb_ref,
             he_ref, p_ref, cnt2_ref):
    oh = oh_ref[...]
    am = am_ref[...]
    m = m_ref[...]
    m2p = m2p_ref[...]
    pos = pos_ref[...]
    h0 = jnp.dot(oh, am)
    cnt = jnp.maximum(jnp.sum(m, axis=1, keepdims=True), 1.0)
    h_suf = jnp.dot(m, h0) / cnt
    p_suf = jnp.dot(m, pos) / cnt
    hcat = h0 + jnp.dot(m2p, h_suf)
    pcat = pos + jnp.dot(m2p, p_suf)
    he_ref[...] = jnp.dot(hcat, wemb_ref[...]) + bemb_ref[...]
    p_ref[...] = pcat
    cnt2_ref[...] = cnt + 1.0


def _k1_body(h_ref, p_ref, w1a_ref, w1b_ref, b1_ref, t1_ref, t2_ref):
    h = h_ref[...]
    t1_ref[...] = jnp.dot(h, w1a_ref[...]) + b1_ref[...]
    t2_ref[...] = jnp.dot(h, w1b_ref[...])
    del p_ref


def _expand_col(colM):
    r = lax.broadcasted_iota(jnp.int32, (BK, RPB), 0) // 128
    k = lax.broadcasted_iota(jnp.int32, (BK, RPB), 1)
    o1 = (r == k).astype(jnp.float32)
    g = jnp.dot(o1, colM)
    lm = lax.broadcasted_iota(jnp.int32, (BK, 128), 0) % 128
    ll = lax.broadcasted_iota(jnp.int32, (BK, 128), 1)
    sel = (ll == lm).astype(jnp.float32)
    return jnp.sum(g * sel, axis=1, keepdims=True)


def _compress_col(col):
    r = lax.broadcasted_iota(jnp.int32, (BK, RPB), 0) // 128
    k = lax.broadcasted_iota(jnp.int32, (BK, RPB), 1)
    o1 = (r == k).astype(jnp.float32)
    lm = lax.broadcasted_iota(jnp.int32, (BK, 128), 0) % 128
    ll = lax.broadcasted_iota(jnp.int32, (BK, 128), 1)
    sel = (ll == lm).astype(jnp.float32)
    return jnp.dot(o1.T, sel * col)


def _k3_body(xi_ref, xj_ref, distM_ref, w1c_ref, m0_ref, s1_ref, s2_ref):
    i = pl.program_id(0)
    dist = _expand_col(distM_ref[...])
    z = xi_ref[...] + xj_ref[...] + dist * w1c_ref[...]
    m0 = z * jax.nn.sigmoid(z)
    m0_ref[...] = m0
    gidx = i * BK + lax.broadcasted_iota(jnp.int32, (BK, 1), 0)
    mask = (gidx < ET).astype(jnp.float32)
    mm = m0 * mask

    @pl.when(i == 0)
    def _():
        s1_ref[...] = jnp.zeros_like(s1_ref)
        s2_ref[...] = jnp.zeros_like(s2_ref)

    s1_ref[...] += jnp.sum(mm, axis=0, keepdims=True)
    s2_ref[...] += jnp.sum(mm * m0, axis=0, keepdims=True)


def _k4_body(m0_ref, dvxM_ref, dvyM_ref, dvzM_ref, s1_ref, s2_ref,
             g_ref, bb_ref, w2_ref, b2_ref, wi_ref, bi_ref,
             wp1_ref, bp1_ref, wp2_ref, bp2_ref,
             msg_ref, pmxM_ref, pmyM_ref, pmzM_ref):
    mu = s1_ref[...] / ET
    var = s2_ref[...] / ET - mu * mu
    mn = (m0_ref[...] - mu) / jnp.sqrt(var + 1e-5) * g_ref[...] + bb_ref[...]
    m1 = _swish(jnp.dot(mn, w2_ref[...]) + b2_ref[...])
    t = jnp.sum(m1 * wi_ref[...], axis=1, keepdims=True) + bi_ref[...]
    m2 = jax.nn.sigmoid(t) * m1
    u = _swish(jnp.dot(m2, wp1_ref[...]) + bp1_ref[...])
    q = jnp.sum(u * wp2_ref[...], axis=1, keepdims=True) + bp2_ref[...]
    msg_ref[...] = m2
    pmxM_ref[...] = _compress_col(_expand_col(dvxM_ref[...]) * q)
    pmyM_ref[...] = _compress_col(_expand_col(dvyM_ref[...]) * q)
    pmzM_ref[...] = _compress_col(_expand_col(dvzM_ref[...]) * q)


NB = 2504
NNB = NP // NB


def _k6a_body(h_ref, p_ref, a0_ref, a1_ref, s0_ref, s16_1_ref, tagsf_ref,
              m2t_ref, wu1a_ref, wu1b_ref, bu1_ref, wu2_ref, bu2_ref,
              h2_ref, pn_ref, sgm_ref, sgv_ref):
    i = pl.program_id(0)
    aggm = a0_ref[...] + a1_ref[...]
    a16 = s0_ref[...] + s16_1_ref[...]
    deg = jnp.maximum(a16[:, 3:4], 1.0)
    aggm = aggm / deg
    pupd = a16[:, 0:4] / deg
    colmask = (lax.broadcasted_iota(jnp.int32, (NB, 4), 1) < 3).astype(jnp.float32)
    h = h_ref[...]
    pn_ref[...] = p_ref[...] + pupd * colmask * tagsf_ref[...]
    u = _swish(jnp.dot(h, wu1a_ref[...]) + jnp.dot(aggm, wu1b_ref[...]) + bu1_ref[...])
    h2 = h + _swish(jnp.dot(u, wu2_ref[...]) + bu2_ref[...])
    h2_ref[...] = h2

    @pl.when(i == 0)
    def _():
        sgm_ref[...] = jnp.zeros_like(sgm_ref)
        sgv_ref[...] = jnp.zeros_like(sgv_ref)

    m2t = m2t_ref[...]
    dn = (((0,), (0,)), ((), ()))
    sgm_ref[...] += lax.dot_general(m2t, h2, dn)
    sgv_ref[...] += lax.dot_general(m2t, h2 * h2, dn)


def _k6b_body(h2_ref, m2t_ref, sgm_ref, sgv_ref, cnt2_ref, hn_ref):
    i = pl.program_id(0)
    cnt2 = cnt2_ref[...]
    gm = sgm_ref[...] / cnt2
    gv = sgv_ref[...] / cnt2 - gm * gm
    m2t = m2t_ref[...]
    bgm = jnp.dot(m2t, gm)
    bgv = jnp.dot(m2t, gv)
    h2 = h2_ref[...]
    gidx = i * NB + lax.broadcasted_iota(jnp.int32, (NB, 1), 0)
    valid = (gidx < NN).astype(jnp.float32)
    hn_ref[...] = (h2 - bgm) / jnp.sqrt(jnp.maximum(bgv, 0.0) + 1e-5) * valid


_SC_MESH = plsc.VectorSubcoreMesh(core_axis_name="c", subcore_axis_name="s")


def _k2_body(t1_hbm, t2_hbm, dst_hbm, src_hbm, cox_hbm, coy_hbm, coz_hbm,
             px_hbm, py_hbm, pz_hbm,
             xi_hbm, xj_hbm, dvx_hbm, dvy_hbm, dvz_hbm, dist_hbm,
             idxd_v, idxs_v, rowsi_v, rowsj_v, pxv, pyv, pzv,
             coxv, coyv, cozv, dvxv, dvyv, dvzv, distv, sem1, sem2):
    c = lax.axis_index("c")
    s = lax.axis_index("s")
    wid = s * 2 + c
    base = wid * EPW
    pltpu.sync_copy(px_hbm, pxv)
    pltpu.sync_copy(py_hbm, pyv)
    pltpu.sync_copy(pz_hbm, pzv)

    def body(it, carry):
        off = base + it * CH
        pltpu.sync_copy(dst_hbm.at[pl.ds(off, CH)], idxd_v)
        pltpu.sync_copy(src_hbm.at[pl.ds(off, CH)], idxs_v)
        pltpu.sync_copy(cox_hbm.at[pl.ds(off, CH)], coxv)
        pltpu.sync_copy(coy_hbm.at[pl.ds(off, CH)], coyv)
        pltpu.sync_copy(coz_hbm.at[pl.ds(off, CH)], cozv)
        cpi = pltpu.async_copy(t1_hbm.at[idxd_v], rowsi_v, sem1)
        cpj = pltpu.async_copy(t2_hbm.at[idxs_v], rowsj_v, sem2)

        def grp(j, carry2):
            sl = pl.ds(j * 16, 16)
            ids = idxd_v[sl]
            iss = idxs_v[sl]
            dx = (plsc.load_gather(pxv, [ids]) - plsc.load_gather(pxv, [iss])
                  + coxv[sl])
            dy = (plsc.load_gather(pyv, [ids]) - plsc.load_gather(pyv, [iss])
                  + coyv[sl])
            dz = (plsc.load_gather(pzv, [ids]) - plsc.load_gather(pzv, [iss])
                  + cozv[sl])
            dvxv[sl] = dx
            dvyv[sl] = dy
            dvzv[sl] = dz
            distv[sl] = dx * dx + dy * dy + dz * dz
            return carry2

        lax.fori_loop(0, CH // 16, grp, 0, unroll=False)
        cpi.wait()
        cpj.wait()
        pltpu.sync_copy(rowsi_v, xi_hbm.at[pl.ds(off, CH)])
        pltpu.sync_copy(rowsj_v, xj_hbm.at[pl.ds(off, CH)])
        pltpu.sync_copy(dvxv, dvx_hbm.at[pl.ds(off, CH)])
        pltpu.sync_copy(dvyv, dvy_hbm.at[pl.ds(off, CH)])
        pltpu.sync_copy(dvzv, dvz_hbm.at[pl.ds(off, CH)])
        pltpu.sync_copy(distv, dist_hbm.at[pl.ds(off, CH)])
        return carry

    lax.fori_loop(0, NCH, body, 0, unroll=False)


_k2_call = pl.kernel(
    _k2_body,
    out_type=[_sds((ETP, 128)), _sds((ETP, 128)),
              _sds((ETP,)), _sds((ETP,)), _sds((ETP,)), _sds((ETP,))],
    mesh=_SC_MESH,
    scratch_types=[
        pltpu.VMEM((CH,), jnp.int32), pltpu.VMEM((CH,), jnp.int32),
        pltpu.VMEM((CH, 128), jnp.float32), pltpu.VMEM((CH, 128), jnp.float32),
        pltpu.VMEM((NP,), jnp.float32), pltpu.VMEM((NP,), jnp.float32),
        pltpu.VMEM((NP,), jnp.float32),
        pltpu.VMEM((CH,), jnp.float32), pltpu.VMEM((CH,), jnp.float32),
        pltpu.VMEM((CH,), jnp.float32),
        pltpu.VMEM((CH,), jnp.float32), pltpu.VMEM((CH,), jnp.float32),
        pltpu.VMEM((CH,), jnp.float32), pltpu.VMEM((CH,), jnp.float32),
        pltpu.SemaphoreType.DMA, pltpu.SemaphoreType.DMA,
    ],
    compiler_params=pltpu.CompilerParams(needs_layout_passes=False),
)


def _k5_body(msg_hbm, dst_hbm, z128_hbm, aggm_hbm, shm, rowsm_v, idxv):
    c = lax.axis_index("c")
    s = lax.axis_index("s")
    base = s * (ETP // 16)

    @pl.when(s == 0)
    def _():
        pltpu.sync_copy(z128_hbm, shm)

    plsc.subcore_barrier()

    lane = lax.iota(jnp.int32, 16)
    coff = c * HALF
    dummy = HALF + (lane & 7)

    def body(it, carry):
        off = base + it * CH
        pltpu.sync_copy(msg_hbm.at[pl.ds(off, CH)], rowsm_v)
        pltpu.sync_copy(dst_hbm.at[pl.ds(off, 128)], idxv.at[0])
        pltpu.sync_copy(dst_hbm.at[pl.ds(off + 128, 128)], idxv.at[1])

        def grp(j, carry2):
            r = j >> 3
            sl2 = pl.ds((j & 7) * 16, 16)
            loc = idxv[r, sl2] - coff
            oob = (loc < 0) | (loc >= HALF)
            idxv[r, sl2] = jnp.where(oob, dummy, loc)
            return carry2

        lax.fori_loop(0, CH // 16, grp, 0, unroll=False)
        pltpu.sync_copy(rowsm_v.at[pl.ds(0, 128)], shm.at[idxv.at[0]], add=True)
        pltpu.sync_copy(rowsm_v.at[pl.ds(128, 128)], shm.at[idxv.at[1]], add=True)
        return carry

    lax.fori_loop(0, ETP // 16 // CH, body, 0, unroll=False)
    plsc.subcore_barrier()

    @pl.when(s == 0)
    def _():
        pltpu.sync_copy(shm, aggm_hbm.at[c])


_k5_call = pl.kernel(
    _k5_body,
    out_type=_sds((2, HALF_P, 128)),
    mesh=_SC_MESH,
    scratch_types=[
        pltpu.VMEM_SHARED((HALF_P, 128), jnp.float32),
        pltpu.VMEM((CH, 128), jnp.float32),
        pltpu.VMEM((2, 128), jnp.int32),
    ],
    compiler_params=pltpu.CompilerParams(needs_layout_passes=False),
)


def _k5b_body(pmx_hbm, pmy_hbm, pmz_hbm, dst_hbm, z128_hbm, agg16_hbm,
              shm, rowsp_v, idxv, pmxv, pmyv, pmzv):
    c = lax.axis_index("c")
    s = lax.axis_index("s")
    base = s * (ETP // 16)

    @pl.when(s == 0)
    def _():
        pltpu.sync_copy(z128_hbm, shm)

    col3 = (lax.iota(jnp.int32, 16) == 3).astype(jnp.float32)
    zero16 = jnp.zeros((16,), jnp.float32)

    def initrow(j, carry):
        rowsp_v[j, pl.ds(0, 16)] = col3
        for k in range(1, 8):
            rowsp_v[j, pl.ds(k * 16, 16)] = zero16
        return carry

    lax.fori_loop(0, CH, initrow, 0, unroll=False)
    plsc.subcore_barrier()

    lane = lax.iota(jnp.int32, 16)
    coff = c * HALF
    dummy = HALF + (lane & 7)

    def body(it, carry):
        off = base + it * CH
        pltpu.sync_copy(dst_hbm.at[pl.ds(off, 128)], idxv.at[0])
        pltpu.sync_copy(dst_hbm.at[pl.ds(off + 128, 128)], idxv.at[1])
        pltpu.sync_copy(pmx_hbm.at[pl.ds(off, CH)], pmxv)
        pltpu.sync_copy(pmy_hbm.at[pl.ds(off, CH)], pmyv)
        pltpu.sync_copy(pmz_hbm.at[pl.ds(off, CH)], pmzv)

        def grp(j, carry2):
            sl = pl.ds(j * 16, 16)
            rows = j * 16 + lane
            plsc.store_scatter(rowsp_v, [rows, jnp.zeros((16,), jnp.int32)],
                               pmxv[sl])
            plsc.store_scatter(rowsp_v, [rows, jnp.ones((16,), jnp.int32)],
                               pmyv[sl])
            plsc.store_scatter(rowsp_v, [rows, jnp.full((16,), 2, jnp.int32)],
                               pmzv[sl])
            r = j >> 3
            sl2 = pl.ds((j & 7) * 16, 16)
            loc = idxv[r, sl2] - coff
            oob = (loc < 0) | (loc >= HALF)
            idxv[r, sl2] = jnp.where(oob, dummy, loc)
            return carry2

        lax.fori_loop(0, CH // 16, grp, 0, unroll=False)
        pltpu.sync_copy(rowsp_v.at[pl.ds(0, 128)], shm.at[idxv.at[0]], add=True)
        pltpu.sync_copy(rowsp_v.at[pl.ds(128, 128)], shm.at[idxv.at[1]], add=True)
        return carry

    lax.fori_loop(0, ETP // 16 // CH, body, 0, unroll=False)
    plsc.subcore_barrier()

    @pl.when(s == 0)
    def _():
        pltpu.sync_copy(shm, agg16_hbm.at[c])


_k5b_call = pl.kernel(
    _k5b_body,
    out_type=_sds((2, HALF_P, 128)),
    mesh=_SC_MESH,
    scratch_types=[
        pltpu.VMEM_SHARED((HALF_P, 128), jnp.float32),
        pltpu.VMEM((CH, 128), jnp.float32),
        pltpu.VMEM((2, 128), jnp.int32),
        pltpu.VMEM((CH,), jnp.float32), pltpu.VMEM((CH,), jnp.float32),
        pltpu.VMEM((CH,), jnp.float32),
    ],
    compiler_params=pltpu.CompilerParams(needs_layout_passes=False),
)


def _k7_body(e_ref, w1_ref, b1_ref, w2_ref, b2_ref, w3_ref, b3_ref, o_ref):
    e = e_ref[...]
    e = _swish(jnp.dot(e, w1_ref[...]) + b1_ref[...])
    e = _swish(jnp.dot(e, w2_ref[...]) + b2_ref[...])
    o_ref[...] = jnp.dot(e, w3_ref[...]) + b3_ref[...]


def kernel(atomic_numbers, pos, edge_index, cell_offsets, tags, batch, atom_map, params):
    idt = edge_index.dtype
    edge_i = N + batch
    edge_j = jnp.arange(N, dtype=idt)
    src = jnp.concatenate([edge_index[0], edge_i, edge_j])
    dst = jnp.concatenate([edge_index[1], edge_j, edge_i])
    pad_e = ETP - ET
    dst_p = jnp.concatenate([dst, jnp.full((pad_e,), NN, idt)]).astype(jnp.int32)
    src_p = jnp.concatenate([src, jnp.full((pad_e,), NN, idt)]).astype(jnp.int32)
    co = jnp.concatenate([cell_offsets, jnp.zeros((2 * N, 3), _f32)], 0)
    co_p = jnp.concatenate([co, jnp.zeros((pad_e, 3), _f32)], 0)
    cox, coy, coz = co_p[:, 0], co_p[:, 1], co_p[:, 2]

    oh = (atomic_numbers[:, None] == jnp.arange(104)[None, :]).astype(_f32)
    oh = jnp.concatenate([oh, jnp.zeros((NP - N, 104), _f32)], 0)
    amp = jnp.zeros((104, 16), _f32).at[:101, :IN_F].set(atom_map)
    gids = jnp.arange(B, dtype=batch.dtype)
    m_real = (batch[None, :] == gids[:, None]).astype(_f32)
    m_mat = jnp.concatenate([m_real, jnp.zeros((B, NP - N), _f32)], 1)
    phant = jnp.concatenate(
        [jnp.zeros((B, N), _f32), jnp.eye(B, dtype=_f32),
         jnp.zeros((B, NP - NN), _f32)], 1)
    m2_mat = m_mat + phant
    m2p = phant.T
    pos_p = jnp.zeros((NP, 4), _f32).at[:N, :3].set(pos)
    tagsf = jnp.concatenate(
        [(tags == 2).astype(_f32), jnp.ones((B,), _f32),
         jnp.zeros((NP - NN,), _f32)])[:, None]
    wemb, bemb = params['emb']
    wemb_p = jnp.zeros((16, 128), _f32).at[:IN_F].set(wemb)
    z128 = jnp.zeros((HALF_P, 128), _f32)

    he, p, cnt2 = pl.pallas_call(
        _k0_body,
        out_shape=[_sds((NP, 128)), _sds((NP, 4)), _sds((B, 1))],
    )(oh, amp, m_mat, m2p, pos_p, wemb_p, bemb[None, :])

    h = he
    for lp in params['layers']:
        w1, b1 = lp['msg1']
        w1a, w1b, w1c = w1[:128], w1[128:256], w1[256:257]
        t1, t2 = pl.pallas_call(
            _k1_body,
            out_shape=[_sds((NP, 128)), _sds((NP, 128))],
        )(h, p, w1a, w1b, b1[None, :])

        xi, xj, dvx, dvy, dvz, dist = _k2_call(
            t1, t2, dst_p, src_p, cox, coy, coz,
            p[:, 0], p[:, 1], p[:, 2])
        distM = dist.reshape(2688, 128)
        dvxM = dvx.reshape(2688, 128)
        dvyM = dvy.reshape(2688, 128)
        dvzM = dvz.reshape(2688, 128)

        espec = pl.BlockSpec((BK, 128), lambda i: (i, 0))
        cspec = pl.BlockSpec((RPB, 128), lambda i: (i, 0))
        wspec = pl.BlockSpec((1, 128), lambda i: (0, 0))
        m0, s1, s2 = pl.pallas_call(
            _k3_body,
            grid=(NBK,),
            in_specs=[espec, espec, cspec, wspec],
            out_specs=[espec, wspec, wspec],
            out_shape=[_sds((ETP, 128)), _sds((1, 128)), _sds((1, 128))],
        )(xi, xj, distM, w1c)

        g, bb = lp['bn']
        w2, b2 = lp['msg2']
        wi, bi = lp['inf']
        wp1, bp1 = lp['pos1']
        wp2, bp2 = lp['pos2']
        w128 = pl.BlockSpec((128, 128), lambda i: (0, 0))
        s11 = pl.BlockSpec((1, 1), lambda i: (0, 0))
        msgM, pmxM, pmyM, pmzM = pl.pallas_call(
            _k4_body,
            grid=(NBK,),
            in_specs=[espec, cspec, cspec, cspec, wspec, wspec,
                      wspec, wspec, w128, wspec, wspec, s11,
                      w128, wspec, wspec, s11],
            out_specs=[espec, cspec, cspec, cspec],
            out_shape=[_sds((ETP, 128)), _sds((2688, 128)),
                       _sds((2688, 128)), _sds((2688, 128))],
        )(m0, dvxM, dvyM, dvzM, s1, s2, g[None, :], bb[None, :],
          w2, b2[None, :], wi.T, bi[None, :], wp1, bp1[None, :],
          wp2.T, bp2[None, :])

        aggM = _k5_call(msgM, dst_p, z128)
        agg16 = _k5b_call(pmxM.reshape(-1), pmyM.reshape(-1),
                          pmzM.reshape(-1), dst_p, z128)
        aggM0 = jnp.concatenate([aggM[0, :HALF], aggM[1, :HALF]], 0)
        aggM1 = jnp.zeros((NP, 128), _f32)
        agg16_0 = jnp.concatenate([agg16[0, :HALF, :16],
                                   agg16[1, :HALF, :16]], 0)
        agg16_1 = jnp.zeros((NP, 16), _f32)

        wu1, bu1 = lp['upd1']
        wu2, bu2 = lp['upd2']
        nspec = pl.BlockSpec((NB, 128), lambda i: (i, 0))
        n4spec = pl.BlockSpec((NB, 4), lambda i: (i, 0))
        n16spec = pl.BlockSpec((NB, 16), lambda i: (i, 0))
        n1spec = pl.BlockSpec((NB, 1), lambda i: (i, 0))
        g8spec = pl.BlockSpec((8, NB), lambda i: (0, i))
        g8tspec = pl.BlockSpec((NB, 8), lambda i: (i, 0))
        s8spec = pl.BlockSpec((8, 128), lambda i: (0, 0))
        s81spec = pl.BlockSpec((8, 1), lambda i: (0, 0))
        w128s = pl.BlockSpec((128, 128), lambda i: (0, 0))
        w1s = pl.BlockSpec((1, 128), lambda i: (0, 0))
        h2, p, sgm, sgv = pl.pallas_call(
            _k6a_body,
            grid=(NNB,),
            in_specs=[nspec, n4spec, nspec, nspec, n16spec, n16spec,
                      n1spec, g8tspec, w128s, w128s, w1s, w128s, w1s],
            out_specs=[nspec, n4spec, s8spec, s8spec],
            out_shape=[_sds((NP, 128)), _sds((NP, 4)),
                       _sds((8, 128)), _sds((8, 128))],
        )(h, p, aggM0, aggM1, agg16_0, agg16_1, tagsf, m2_mat.T,
          wu1[:128], wu1[128:], bu1[None, :], wu2, bu2[None, :])
        h = pl.pallas_call(
            _k6b_body,
            grid=(NNB,),
            in_specs=[nspec, g8tspec, s8spec, s8spec, s81spec],
            out_specs=nspec,
            out_shape=_sds((NP, 128)),
        )(h2, m2_mat.T, sgm, sgv, cnt2)

    e = h[N:NN]
    (w1, b1), (w2, b2), (w3, b3) = params['e1'], params['e2'], params['e3']
    return pl.pallas_call(
        _k7_body,
        out_shape=jax.ShapeDtypeStruct((B, 1), _f32),
    )(e, w1, b1[None, :], w2, b2[None, :], w3, b3[None, :])
